# baseline (speedup 1.0000x reference)
import jax
import jax.numpy as jnp
from jax import lax
from jax.experimental import pallas as pl
from jax.experimental.pallas import tpu as pltpu
from jax.experimental.pallas import tpu_sc as plsc

N = 10000
E = 320000
H = 8
C = 16
F = 128
ED = 16
OUT = 16

NC = 2
NS = 16
NW = NC * NS
K = 128
EP = 327680
EW = EP // NW
CH = EW // K
NP = N + 16
ZR = NP // NS
ZB = 64

_mesh = plsc.VectorSubcoreMesh(core_axis_name="c", subcore_axis_name="s",
                               num_cores=NC, num_subcores=NS)

_sc_params = pltpu.CompilerParams(use_tc_tiling_on_sc=False)

_f32 = jnp.float32


def _lrelu(v):
    return jnp.where(v >= 0.0, v, 0.2 * v)


def _zero_stripe(zb, sp_ref, base_r):
    for k in range(ZR // ZB):
        pltpu.sync_copy(zb, sp_ref.at[pl.ds(base_r + k * ZB, ZB), :])
    rem = ZR - (ZR // ZB) * ZB
    pltpu.sync_copy(zb.at[pl.ds(0, rem)],
                    sp_ref.at[pl.ds(base_r + (ZR // ZB) * ZB, rem), :])


def _splat(v, h):
    dn = lax.GatherDimensionNumbers(offset_dims=(), collapsed_slice_dims=(0,),
                                    start_index_map=(0,))
    idx = jnp.full((16, 1), h, jnp.int32)
    return lax.gather(v, idx, dn, (1,),
                      mode=lax.GatherScatterMode.PROMISE_IN_BOUNDS)


def _make_pass1(with_cnt, lcol):
    W = 48 if with_cnt else 32
    out_type = (jax.ShapeDtypeStruct((EP, 16), _f32),
                jax.ShapeDtypeStruct((NC, NP, W), _f32))
    scratch = [pltpu.VMEM((2, 2, K), jnp.int32),
               pltpu.VMEM((2, K, 16), _f32),
               pltpu.VMEM((2, K, 16), _f32),
               pltpu.VMEM((2, K, W), _f32),
               pltpu.VMEM((ZB, W), _f32),
               pltpu.SemaphoreType.DMA,
               pltpu.SemaphoreType.DMA,
               pltpu.SemaphoreType.DMA,
               pltpu.SemaphoreType.DMA,
               pltpu.SemaphoreType.DMA,
               pltpu.SemaphoreType.DMA,
               pltpu.VMEM_SHARED((NP, W), _f32)]

    def body(src2, a16, d16p, ae, exo, acco,
             sdv2, sav, sdv, cbuf, zb, sa0, sa1, sd0, sd1, se0, se1, acc_sp):
        semA = (sa0, sa1)
        semD = (sd0, sd1)
        semE = (se0, se1)
        c = lax.axis_index("c")
        s = lax.axis_index("s")
        wid = c * NS + s

        def zb_body(i, carry):
            for k in range(W // 16):
                zb[i, pl.ds(16 * k, 16)] = jnp.zeros((16,), _f32)
            return carry
        lax.fori_loop(0, ZB, zb_body, 0)
        if with_cnt:
            def ones_body(i, carry):
                cbuf[0, i, pl.ds(32, 16)] = jnp.ones((16,), _f32)
                cbuf[1, i, pl.ds(32, 16)] = jnp.ones((16,), _f32)
                return carry
            lax.fori_loop(0, K, ones_body, 0)
        _zero_stripe(zb, acc_sp, s * ZR)
        plsc.subcore_barrier()

        def linear(j, b):
            base = (j * NW + wid) * K
            pltpu.sync_copy(src2.at[:, pl.ds(base, K)], sdv2.at[b])
            pltpu.sync_copy(ae.at[pl.ds(base, K), pl.ds(lcol, 16)],
                            cbuf.at[b, :, pl.ds(16, 16)])

        def gather_issue(b):
            pltpu.async_copy(a16.at[sdv2.at[b, 0]], sav.at[b], semA[b])
            pltpu.async_copy(d16p.at[sdv2.at[b, 1]], sdv.at[b], semD[b])

        def gather_wait(b):
            pltpu.make_async_copy(a16.at[sdv2.at[b, 0]], sav.at[b], semA[b]).wait()
            pltpu.make_async_copy(d16p.at[sdv2.at[b, 1]], sdv.at[b], semD[b]).wait()

        def compute(b):
            def ebody(e, carry2):
                v = sav[b, e] + sdv[b, e] + cbuf[b, e, pl.ds(16, 16)]
                cbuf[b, e, pl.ds(0, 16)] = jnp.exp(_lrelu(v))
                return carry2
            lax.fori_loop(0, K, ebody, 0, unroll=4)

        def ex_issue(j, b):
            base = (j * NW + wid) * K
            pltpu.async_copy(cbuf.at[b, :, pl.ds(0, 16)],
                             exo.at[pl.ds(base, K), :], semE[b])

        def ex_wait(j, b):
            base = (j * NW + wid) * K
            pltpu.make_async_copy(cbuf.at[b, :, pl.ds(0, 16)],
                                  exo.at[pl.ds(base, K), :], semE[b]).wait()

        def scatter_sync(b):
            pltpu.sync_copy(cbuf.at[b], acc_sp.at[sdv2.at[b, 1]], add=True)

        linear(0, 0)
        gather_issue(0)
        linear(1, 1)
        gather_issue(1)

        def superstep(i, carry):
            j0 = 2 * i
            j1 = j0 + 1
            gather_wait(0)

            @pl.when(i > 0)
            def _():
                ex_wait(j0 - 2, 0)
            compute(0)
            ex_issue(j0, 0)
            scatter_sync(0)

            @pl.when(i < CH // 2 - 1)
            def _():
                linear(j0 + 2, 0)
                gather_issue(0)
            gather_wait(1)

            @pl.when(i > 0)
            def _():
                ex_wait(j1 - 2, 1)
            compute(1)
            ex_issue(j1, 1)
            scatter_sync(1)

            @pl.when(i < CH // 2 - 1)
            def _():
                linear(j1 + 2, 1)
                gather_issue(1)
            return carry
        lax.fori_loop(0, CH // 2, superstep, 0)
        ex_wait(CH - 2, 0)
        ex_wait(CH - 1, 1)
        plsc.subcore_barrier()

        @pl.when(s == 0)
        def _():
            pltpu.sync_copy(acc_sp, acco.at[c])

    return pl.kernel(body, out_type=out_type, mesh=_mesh,
                     scratch_types=tuple(scratch), compiler_params=_sc_params)


_pass1_l1 = _make_pass1(True, 0)
_pass1_l2 = _make_pass1(False, 16)


def _pass2_body(src2, ex, xl, outo,
                sdv2, exv, Xv, zb, sg0, sg1, out_sp):
    semG = (sg0, sg1)
    c = lax.axis_index("c")
    s = lax.axis_index("s")
    wid = c * NS + s

    def zb_body(i, carry):
        for k in range(8):
            zb[i, pl.ds(k * 16, 16)] = jnp.zeros((16,), _f32)
        return carry
    lax.fori_loop(0, ZB, zb_body, 0)
    _zero_stripe(zb, out_sp, s * ZR)
    plsc.subcore_barrier()

    def linear(j, b):
        base = (j * NW + wid) * K
        pltpu.sync_copy(src2.at[:, pl.ds(base, K)], sdv2.at[b])
        pltpu.sync_copy(ex.at[pl.ds(base, K), :], exv.at[b])

    def gather_issue(b):
        pltpu.async_copy(xl.at[sdv2.at[b, 0]], Xv.at[b], semG[b])

    def gather_wait(b):
        pltpu.make_async_copy(xl.at[sdv2.at[b, 0]], Xv.at[b], semG[b]).wait()

    def compute(b):
        def ebody(e, carry2):
            ev = exv[b, e]
            for h in range(H):
                sl = pl.ds(h * 16, 16)
                Xv[b, e, sl] = Xv[b, e, sl] * _splat(ev, h)
            return carry2
        lax.fori_loop(0, K, ebody, 0, unroll=2)

    def scatter_sync(b):
        pltpu.sync_copy(Xv.at[b], out_sp.at[sdv2.at[b, 1]], add=True)

    linear(0, 0)
    gather_issue(0)
    linear(1, 1)
    gather_issue(1)

    def superstep(i, carry):
        j0 = 2 * i
        j1 = j0 + 1
        gather_wait(0)
        compute(0)
        scatter_sync(0)

        @pl.when(i < CH // 2 - 1)
        def _():
            linear(j0 + 2, 0)
            gather_issue(0)
        gather_wait(1)
        compute(1)
        scatter_sync(1)

        @pl.when(i < CH // 2 - 1)
        def _():
            linear(j1 + 2, 1)
            gather_issue(1)
        return carry
    lax.fori_loop(0, CH // 2, superstep, 0)
    plsc.subcore_barrier()

    @pl.when(s == 0)
    def _():
        pltpu.sync_copy(out_sp, outo.at[c])


_pass2 = pl.kernel(
    _pass2_body,
    out_type=jax.ShapeDtypeStruct((NC, NP, F), _f32),
    mesh=_mesh,
    scratch_types=(pltpu.VMEM((2, 2, K), jnp.int32),
                   pltpu.VMEM((2, K, 16), _f32),
                   pltpu.VMEM((2, K, F), _f32),
                   pltpu.VMEM((ZB, F), _f32),
                   pltpu.SemaphoreType.DMA,
                   pltpu.SemaphoreType.DMA,
                   pltpu.VMEM_SHARED((NP, F), _f32)),
    compiler_params=_sc_params)


_B = 2000
_EB = 8192


def _premix1_body(x_ref, wcat_ref, xl_ref, a_ref, d_ref):
    p = jnp.dot(x_ref[...], wcat_ref[...], preferred_element_type=_f32)
    xl_ref[...] = p[:, :F]
    a_ref[...] = p[:, F:F + 16]
    d_ref[...] = p[:, F + 16:F + 32]


def _edgepre_body(ea_ref, aecat_ref, out_ref):
    out_ref[...] = jnp.dot(ea_ref[...], aecat_ref[...],
                           preferred_element_type=_f32)


def _make_combine_body(with_cnt):
    def body(outp, acc, acc1, a16, d16, xl, rep, bvec, hraw, stats):
        step = pl.program_id(0)
        den8 = acc[0, :, 0:8] + acc[1, :, 0:8]
        es8 = acc[0, :, 16:24] + acc[1, :, 16:24]
        if with_cnt:
            cnt8 = acc[0, :, 32:40] + acc[1, :, 32:40]
        else:
            cnt8 = acc1[0, :, 32:40] + acc1[1, :, 32:40]
        al = a16[:, :8] + d16[:, :8] + es8 / jnp.maximum(cnt8, 1.0)
        exl = jnp.exp(_lrelu(al))
        dtot = den8 + exl
        o = outp[0] + outp[1]
        hr = (o + xl[...] * jnp.dot(exl, rep[...], preferred_element_type=_f32)) \
            / jnp.dot(dtot, rep[...], preferred_element_type=_f32) + bvec[...]
        hraw[...] = hr
        st = jnp.concatenate([jnp.sum(hr, axis=0, keepdims=True),
                              jnp.sum(hr * hr, axis=0, keepdims=True)], axis=0)

        @pl.when(step == 0)
        def _():
            stats[...] = st

        @pl.when(step > 0)
        def _():
            stats[...] += st
    return body


def _bn_elu(h_blk, stats):
    m = stats[0:1, :] / N
    v = stats[1:2, :] / N - m * m
    xin = (h_blk - m) / jnp.sqrt(v + 1e-5)
    return jnp.where(xin > 0, xin, jnp.exp(xin) - 1.0)


def _premix2_body(hraw, stats, wcat, xl_ref, a_ref, d_ref):
    xin = _bn_elu(hraw[...], stats[...])
    p = jnp.dot(xin, wcat[...], preferred_element_type=_f32)
    xl_ref[...] = p[:, :F]
    a_ref[...] = p[:, F:F + 16]
    d_ref[...] = p[:, F + 16:F + 32]


def _head_body(hraw, stats, wl, blv, out_ref):
    xin = _bn_elu(hraw[...], stats[...])
    out_ref[...] = jnp.dot(xin, wl[...], preferred_element_type=_f32) + blv[...]


def _node_spec(width):
    return pl.BlockSpec((_B, width), lambda i: (i, 0))


def _full_spec(shape):
    return pl.BlockSpec(shape, lambda i: tuple(0 for _ in shape))


_premix = pl.pallas_call(
    _premix1_body,
    grid=(N // _B,),
    in_specs=[_node_spec(F), _full_spec((F, F + 32))],
    out_specs=[_node_spec(F), _node_spec(16), _node_spec(16)],
    out_shape=[jax.ShapeDtypeStruct((N, F), _f32),
               jax.ShapeDtypeStruct((N, 16), _f32),
               jax.ShapeDtypeStruct((N, 16), _f32)],
)

_edgepre = pl.pallas_call(
    _edgepre_body,
    grid=(EP // _EB,),
    in_specs=[pl.BlockSpec((_EB, ED), lambda i: (i, 0)), _full_spec((ED, 32))],
    out_specs=pl.BlockSpec((_EB, 32), lambda i: (i, 0)),
    out_shape=jax.ShapeDtypeStruct((EP, 32), _f32),
)


def _make_combine(with_cnt):
    w = 48 if with_cnt else 32
    return pl.pallas_call(
        _make_combine_body(with_cnt),
        grid=(N // _B,),
        in_specs=[pl.BlockSpec((NC, _B, F), lambda i: (0, i, 0)),
                  pl.BlockSpec((NC, _B, w), lambda i: (0, i, 0)),
                  pl.BlockSpec((NC, _B, 48), lambda i: (0, i, 0)),
                  _node_spec(16), _node_spec(16), _node_spec(F),
                  _full_spec((H, F)), _full_spec((1, F))],
        out_specs=[_node_spec(F), _full_spec((2, F))],
        out_shape=[jax.ShapeDtypeStruct((N, F), _f32),
                   jax.ShapeDtypeStruct((2, F), _f32)],
    )


_combine1 = _make_combine(True)
_combine2 = _make_combine(False)

_premix_next = pl.pallas_call(
    _premix2_body,
    grid=(N // _B,),
    in_specs=[_node_spec(F), _full_spec((2, F)), _full_spec((F, F + 32))],
    out_specs=[_node_spec(F), _node_spec(16), _node_spec(16)],
    out_shape=[jax.ShapeDtypeStruct((N, F), _f32),
               jax.ShapeDtypeStruct((N, 16), _f32),
               jax.ShapeDtypeStruct((N, 16), _f32)],
)

_head = pl.pallas_call(
    _head_body,
    grid=(N // _B,),
    in_specs=[_node_spec(F), _full_spec((2, F)), _full_spec((F, OUT)),
              _full_spec((1, OUT))],
    out_specs=_node_spec(OUT),
    out_shape=jax.ShapeDtypeStruct((N, OUT), _f32),
)


def _fold(W, att):
    a = (W.reshape(W.shape[0], H, C) * att[None]).sum(-1)
    return jnp.concatenate([a, a], axis=1)


def kernel(x, edge_index, edge_attr, W1, as1, ad1, We1, ae1, b1,
           W2, as2, ad2, We2, ae2, b2, Wl, bl):
    src = edge_index[0]
    dst = edge_index[1]
    pad = EP - E
    src_p = jnp.concatenate([src, jnp.zeros((pad,), jnp.int32)])
    dst_p = jnp.concatenate([dst,
                             N + (jnp.arange(pad, dtype=jnp.int32) % (NP - N))])
    ei_p = jnp.stack([src_p, dst_p])
    ea_p = jnp.concatenate([edge_attr, jnp.zeros((pad, ED), _f32)], axis=0)

    aecat = jnp.concatenate([_fold(We1, ae1), _fold(We2, ae2)], axis=1)
    wcat1 = jnp.concatenate([W1, _fold(W1, as1), _fold(W1, ad1)], axis=1)
    wcat2 = jnp.concatenate([W2, _fold(W2, as2), _fold(W2, ad2)], axis=1)
    rep = jnp.zeros((H, F), _f32)
    rep = rep.at[jnp.repeat(jnp.arange(H), C), jnp.arange(F)].set(1.0)

    AE = _edgepre(ea_p, aecat)
    xl1, a1, d1 = _premix(x, wcat1)
    d1p = jnp.concatenate([d1, jnp.zeros((NP - N, 16), _f32)], axis=0)

    ex1, acc1 = _pass1_l1(ei_p, a1, d1p, AE)
    out1 = _pass2(ei_p, ex1, xl1)
    h1, st1 = _combine1(out1, acc1, acc1, a1, d1, xl1, rep, b1.reshape(1, F))

    xl2, a2, d2 = _premix_next(h1, st1, wcat2)
    d2p = jnp.concatenate([d2, jnp.zeros((NP - N, 16), _f32)], axis=0)

    ex2, acc2 = _pass1_l2(ei_p, a2, d2p, AE)
    out2 = _pass2(ei_p, ex2, xl2)
    h2, st2 = _combine2(out2, acc2, acc1, a2, d2, xl2, rep, b2.reshape(1, F))

    return _head(h2, st2, Wl, bl.reshape(1, OUT))


# speedup vs baseline: 1.0784x; 1.0784x over previous
import jax
import jax.numpy as jnp
from jax import lax
from jax.experimental import pallas as pl
from jax.experimental.pallas import tpu as pltpu
from jax.experimental.pallas import tpu_sc as plsc

N = 10000
E = 320000
H = 8
C = 16
F = 128
ED = 16
OUT = 16

NC = 2
NS = 16
NW = NC * NS
K = 128
EP = 327680
EW = EP // NW
CH = EW // K
NP = N + 16
ZR = NP // NS
ZB = 64

_mesh = plsc.VectorSubcoreMesh(core_axis_name="c", subcore_axis_name="s",
                               num_cores=NC, num_subcores=NS)

_sc_params = pltpu.CompilerParams(use_tc_tiling_on_sc=False)

_f32 = jnp.float32


def _lrelu(v):
    return jnp.where(v >= 0.0, v, 0.2 * v)


def _zero_stripe(zb, sp_ref, base_r):
    for k in range(ZR // ZB):
        pltpu.sync_copy(zb, sp_ref.at[pl.ds(base_r + k * ZB, ZB), :])
    rem = ZR - (ZR // ZB) * ZB
    pltpu.sync_copy(zb.at[pl.ds(0, rem)],
                    sp_ref.at[pl.ds(base_r + (ZR // ZB) * ZB, rem), :])


def _splat(v, h):
    dn = lax.GatherDimensionNumbers(offset_dims=(), collapsed_slice_dims=(0,),
                                    start_index_map=(0,))
    idx = jnp.full((16, 1), h, jnp.int32)
    return lax.gather(v, idx, dn, (1,),
                      mode=lax.GatherScatterMode.PROMISE_IN_BOUNDS)


def _make_pass1(with_cnt, lcol):
    W = 48 if with_cnt else 32
    out_type = (jax.ShapeDtypeStruct((EP, 16), _f32),
                jax.ShapeDtypeStruct((NC, NP, W), _f32))
    scratch = [pltpu.VMEM((2, 2, K), jnp.int32),
               pltpu.VMEM((2, K, 16), _f32),
               pltpu.VMEM((2, K, 16), _f32),
               pltpu.VMEM((2, K, W), _f32),
               pltpu.VMEM((ZB, W), _f32),
               pltpu.SemaphoreType.DMA,
               pltpu.SemaphoreType.DMA,
               pltpu.SemaphoreType.DMA,
               pltpu.SemaphoreType.DMA,
               pltpu.SemaphoreType.DMA,
               pltpu.SemaphoreType.DMA,
               pltpu.VMEM_SHARED((NP, W), _f32)]

    def body(src2, a16, d16p, ae, exo, acco,
             sdv2, sav, sdv, cbuf, zb, sa0, sa1, sd0, sd1, se0, se1, acc_sp):
        semA = (sa0, sa1)
        semD = (sd0, sd1)
        semE = (se0, se1)
        c = lax.axis_index("c")
        s = lax.axis_index("s")
        wid = c * NS + s

        def zb_body(i, carry):
            for k in range(W // 16):
                zb[i, pl.ds(16 * k, 16)] = jnp.zeros((16,), _f32)
            return carry
        lax.fori_loop(0, ZB, zb_body, 0)
        if with_cnt:
            def ones_body(i, carry):
                cbuf[0, i, pl.ds(32, 16)] = jnp.ones((16,), _f32)
                cbuf[1, i, pl.ds(32, 16)] = jnp.ones((16,), _f32)
                return carry
            lax.fori_loop(0, K, ones_body, 0)
        _zero_stripe(zb, acc_sp, s * ZR)
        plsc.subcore_barrier()

        def linear(j, b):
            base = (wid * CH + j) * K
            pltpu.sync_copy(src2.at[:, pl.ds(base, K)], sdv2.at[b])
            pltpu.sync_copy(ae.at[pl.ds(base, K), pl.ds(lcol, 16)],
                            cbuf.at[b, :, pl.ds(16, 16)])

        def gather_issue(b):
            pltpu.async_copy(a16.at[sdv2.at[b, 0]], sav.at[b], semA[b])
            pltpu.async_copy(d16p.at[sdv2.at[b, 1]], sdv.at[b], semD[b])

        def gather_wait(b):
            pltpu.make_async_copy(a16.at[sdv2.at[b, 0]], sav.at[b], semA[b]).wait()
            pltpu.make_async_copy(d16p.at[sdv2.at[b, 1]], sdv.at[b], semD[b]).wait()

        def compute(b):
            def ebody(e, carry2):
                v = sav[b, e] + sdv[b, e] + cbuf[b, e, pl.ds(16, 16)]
                cbuf[b, e, pl.ds(0, 16)] = jnp.exp(_lrelu(v))
                return carry2
            lax.fori_loop(0, K, ebody, 0, unroll=4)

        def ex_issue(j, b):
            base = (wid * CH + j) * K
            pltpu.async_copy(cbuf.at[b, :, pl.ds(0, 16)],
                             exo.at[pl.ds(base, K), :], semE[b])

        def ex_wait(j, b):
            base = (wid * CH + j) * K
            pltpu.make_async_copy(cbuf.at[b, :, pl.ds(0, 16)],
                                  exo.at[pl.ds(base, K), :], semE[b]).wait()

        def scatter_sync(b):
            pltpu.sync_copy(cbuf.at[b], acc_sp.at[sdv2.at[b, 1]], add=True)

        linear(0, 0)
        gather_issue(0)
        linear(1, 1)
        gather_issue(1)

        def superstep(i, carry):
            j0 = 2 * i
            j1 = j0 + 1
            gather_wait(0)

            @pl.when(i > 0)
            def _():
                ex_wait(j0 - 2, 0)
            compute(0)
            ex_issue(j0, 0)
            scatter_sync(0)

            @pl.when(i < CH // 2 - 1)
            def _():
                linear(j0 + 2, 0)
                gather_issue(0)
            gather_wait(1)

            @pl.when(i > 0)
            def _():
                ex_wait(j1 - 2, 1)
            compute(1)
            ex_issue(j1, 1)
            scatter_sync(1)

            @pl.when(i < CH // 2 - 1)
            def _():
                linear(j1 + 2, 1)
                gather_issue(1)
            return carry
        lax.fori_loop(0, CH // 2, superstep, 0)
        ex_wait(CH - 2, 0)
        ex_wait(CH - 1, 1)
        plsc.subcore_barrier()

        @pl.when(s == 0)
        def _():
            pltpu.sync_copy(acc_sp, acco.at[c])

    return pl.kernel(body, out_type=out_type, mesh=_mesh,
                     scratch_types=tuple(scratch), compiler_params=_sc_params)


_pass1_l1 = _make_pass1(True, 0)
_pass1_l2 = _make_pass1(False, 16)


def _pass2_body(src2, ex, xl, outo,
                sdv2, exv, Xv, zb, sg0, sg1, out_sp):
    semG = (sg0, sg1)
    c = lax.axis_index("c")
    s = lax.axis_index("s")
    wid = c * NS + s

    def zb_body(i, carry):
        for k in range(8):
            zb[i, pl.ds(k * 16, 16)] = jnp.zeros((16,), _f32)
        return carry
    lax.fori_loop(0, ZB, zb_body, 0)
    _zero_stripe(zb, out_sp, s * ZR)
    plsc.subcore_barrier()

    def linear(j, b):
        base = (wid * CH + j) * K
        pltpu.sync_copy(src2.at[:, pl.ds(base, K)], sdv2.at[b])
        pltpu.sync_copy(ex.at[pl.ds(base, K), :], exv.at[b])

    def gather_issue(b):
        pltpu.async_copy(xl.at[sdv2.at[b, 0]], Xv.at[b], semG[b])

    def gather_wait(b):
        pltpu.make_async_copy(xl.at[sdv2.at[b, 0]], Xv.at[b], semG[b]).wait()

    def compute(b):
        def ebody(e, carry2):
            ev = exv[b, e]
            for h in range(H):
                sl = pl.ds(h * 16, 16)
                Xv[b, e, sl] = Xv[b, e, sl] * _splat(ev, h)
            return carry2
        lax.fori_loop(0, K, ebody, 0, unroll=2)

    def scatter_sync(b):
        pltpu.sync_copy(Xv.at[b], out_sp.at[sdv2.at[b, 1]], add=True)

    linear(0, 0)
    gather_issue(0)
    linear(1, 1)
    gather_issue(1)

    def superstep(i, carry):
        j0 = 2 * i
        j1 = j0 + 1
        gather_wait(0)
        compute(0)
        scatter_sync(0)

        @pl.when(i < CH // 2 - 1)
        def _():
            linear(j0 + 2, 0)
            gather_issue(0)
        gather_wait(1)
        compute(1)
        scatter_sync(1)

        @pl.when(i < CH // 2 - 1)
        def _():
            linear(j1 + 2, 1)
            gather_issue(1)
        return carry
    lax.fori_loop(0, CH // 2, superstep, 0)
    plsc.subcore_barrier()

    @pl.when(s == 0)
    def _():
        pltpu.sync_copy(out_sp, outo.at[c])


_pass2 = pl.kernel(
    _pass2_body,
    out_type=jax.ShapeDtypeStruct((NC, NP, F), _f32),
    mesh=_mesh,
    scratch_types=(pltpu.VMEM((2, 2, K), jnp.int32),
                   pltpu.VMEM((2, K, 16), _f32),
                   pltpu.VMEM((2, K, F), _f32),
                   pltpu.VMEM((ZB, F), _f32),
                   pltpu.SemaphoreType.DMA,
                   pltpu.SemaphoreType.DMA,
                   pltpu.VMEM_SHARED((NP, F), _f32)),
    compiler_params=_sc_params)


_B = 2000
_EB = 8192


def _premix1_body(x_ref, wcat_ref, xl_ref, a_ref, d_ref):
    p = jnp.dot(x_ref[...], wcat_ref[...], preferred_element_type=_f32)
    xl_ref[...] = p[:, :F]
    a_ref[...] = p[:, F:F + 16]
    d_ref[...] = p[:, F + 16:F + 32]


def _edgepre_body(ea_ref, aecat_ref, out_ref):
    out_ref[...] = jnp.dot(ea_ref[...], aecat_ref[...],
                           preferred_element_type=_f32)


def _make_combine_body(with_cnt):
    def body(outp, acc, acc1, a16, d16, xl, rep, bvec, hraw, stats):
        step = pl.program_id(0)
        den8 = acc[0, :, 0:8] + acc[1, :, 0:8]
        es8 = acc[0, :, 16:24] + acc[1, :, 16:24]
        if with_cnt:
            cnt8 = acc[0, :, 32:40] + acc[1, :, 32:40]
        else:
            cnt8 = acc1[0, :, 32:40] + acc1[1, :, 32:40]
        al = a16[:, :8] + d16[:, :8] + es8 / jnp.maximum(cnt8, 1.0)
        exl = jnp.exp(_lrelu(al))
        dtot = den8 + exl
        o = outp[0] + outp[1]
        hr = (o + xl[...] * jnp.dot(exl, rep[...], preferred_element_type=_f32)) \
            / jnp.dot(dtot, rep[...], preferred_element_type=_f32) + bvec[...]
        hraw[...] = hr
        st = jnp.concatenate([jnp.sum(hr, axis=0, keepdims=True),
                              jnp.sum(hr * hr, axis=0, keepdims=True)], axis=0)

        @pl.when(step == 0)
        def _():
            stats[...] = st

        @pl.when(step > 0)
        def _():
            stats[...] += st
    return body


def _bn_elu(h_blk, stats):
    m = stats[0:1, :] / N
    v = stats[1:2, :] / N - m * m
    xin = (h_blk - m) / jnp.sqrt(v + 1e-5)
    return jnp.where(xin > 0, xin, jnp.exp(xin) - 1.0)


def _premix2_body(hraw, stats, wcat, xl_ref, a_ref, d_ref):
    xin = _bn_elu(hraw[...], stats[...])
    p = jnp.dot(xin, wcat[...], preferred_element_type=_f32)
    xl_ref[...] = p[:, :F]
    a_ref[...] = p[:, F:F + 16]
    d_ref[...] = p[:, F + 16:F + 32]


def _head_body(hraw, stats, wl, blv, out_ref):
    xin = _bn_elu(hraw[...], stats[...])
    out_ref[...] = jnp.dot(xin, wl[...], preferred_element_type=_f32) + blv[...]


def _node_spec(width):
    return pl.BlockSpec((_B, width), lambda i: (i, 0))


def _full_spec(shape):
    return pl.BlockSpec(shape, lambda i: tuple(0 for _ in shape))


_premix = pl.pallas_call(
    _premix1_body,
    grid=(N // _B,),
    in_specs=[_node_spec(F), _full_spec((F, F + 32))],
    out_specs=[_node_spec(F), _node_spec(16), _node_spec(16)],
    out_shape=[jax.ShapeDtypeStruct((N, F), _f32),
               jax.ShapeDtypeStruct((N, 16), _f32),
               jax.ShapeDtypeStruct((N, 16), _f32)],
)

_edgepre = pl.pallas_call(
    _edgepre_body,
    grid=(EP // _EB,),
    in_specs=[pl.BlockSpec((_EB, ED), lambda i: (i, 0)), _full_spec((ED, 32))],
    out_specs=pl.BlockSpec((_EB, 32), lambda i: (i, 0)),
    out_shape=jax.ShapeDtypeStruct((EP, 32), _f32),
)


def _make_combine(with_cnt):
    w = 48 if with_cnt else 32
    return pl.pallas_call(
        _make_combine_body(with_cnt),
        grid=(N // _B,),
        in_specs=[pl.BlockSpec((NC, _B, F), lambda i: (0, i, 0)),
                  pl.BlockSpec((NC, _B, w), lambda i: (0, i, 0)),
                  pl.BlockSpec((NC, _B, 48), lambda i: (0, i, 0)),
                  _node_spec(16), _node_spec(16), _node_spec(F),
                  _full_spec((H, F)), _full_spec((1, F))],
        out_specs=[_node_spec(F), _full_spec((2, F))],
        out_shape=[jax.ShapeDtypeStruct((N, F), _f32),
                   jax.ShapeDtypeStruct((2, F), _f32)],
    )


_combine1 = _make_combine(True)
_combine2 = _make_combine(False)

_premix_next = pl.pallas_call(
    _premix2_body,
    grid=(N // _B,),
    in_specs=[_node_spec(F), _full_spec((2, F)), _full_spec((F, F + 32))],
    out_specs=[_node_spec(F), _node_spec(16), _node_spec(16)],
    out_shape=[jax.ShapeDtypeStruct((N, F), _f32),
               jax.ShapeDtypeStruct((N, 16), _f32),
               jax.ShapeDtypeStruct((N, 16), _f32)],
)

_head = pl.pallas_call(
    _head_body,
    grid=(N // _B,),
    in_specs=[_node_spec(F), _full_spec((2, F)), _full_spec((F, OUT)),
              _full_spec((1, OUT))],
    out_specs=_node_spec(OUT),
    out_shape=jax.ShapeDtypeStruct((N, OUT), _f32),
)


def _fold(W, att):
    a = (W.reshape(W.shape[0], H, C) * att[None]).sum(-1)
    return jnp.concatenate([a, a], axis=1)


def kernel(x, edge_index, edge_attr, W1, as1, ad1, We1, ae1, b1,
           W2, as2, ad2, We2, ae2, b2, Wl, bl):
    src = edge_index[0]
    dst = edge_index[1]
    pad = EP - E
    src_p = jnp.concatenate([src, jnp.zeros((pad,), jnp.int32)])
    dst_p = jnp.concatenate([dst,
                             N + (jnp.arange(pad, dtype=jnp.int32) % (NP - N))])
    ei_p = jnp.stack([src_p, dst_p])
    ea_p = jnp.concatenate([edge_attr, jnp.zeros((pad, ED), _f32)], axis=0)

    aecat = jnp.concatenate([_fold(We1, ae1), _fold(We2, ae2)], axis=1)
    wcat1 = jnp.concatenate([W1, _fold(W1, as1), _fold(W1, ad1)], axis=1)
    wcat2 = jnp.concatenate([W2, _fold(W2, as2), _fold(W2, ad2)], axis=1)
    rep = jnp.zeros((H, F), _f32)
    rep = rep.at[jnp.repeat(jnp.arange(H), C), jnp.arange(F)].set(1.0)

    AE = _edgepre(ea_p, aecat)
    xl1, a1, d1 = _premix(x, wcat1)
    d1p = jnp.concatenate([d1, jnp.zeros((NP - N, 16), _f32)], axis=0)

    ex1, acc1 = _pass1_l1(ei_p, a1, d1p, AE)
    out1 = _pass2(ei_p, ex1, xl1)
    h1, st1 = _combine1(out1, acc1, acc1, a1, d1, xl1, rep, b1.reshape(1, F))

    xl2, a2, d2 = _premix_next(h1, st1, wcat2)
    d2p = jnp.concatenate([d2, jnp.zeros((NP - N, 16), _f32)], axis=0)

    ex2, acc2 = _pass1_l2(ei_p, a2, d2p, AE)
    out2 = _pass2(ei_p, ex2, xl2)
    h2, st2 = _combine2(out2, acc2, acc1, a2, d2, xl2, rep, b2.reshape(1, F))

    return _head(h2, st2, Wl, bl.reshape(1, OUT))


# speedup vs baseline: 1.3572x; 1.2585x over previous
import jax
import jax.numpy as jnp
from jax import lax
from jax.experimental import pallas as pl
from jax.experimental.pallas import tpu as pltpu
from jax.experimental.pallas import tpu_sc as plsc

N = 10000
E = 320000
H = 8
C = 16
F = 128
ED = 16
OUT = 16

NC = 2
NS = 16
NW = NC * NS
K = 128
EP = 327680
EW = EP // NW
CH = EW // K
NP = N + 16
ZR = NP // NS
ZB = 64

_mesh = plsc.VectorSubcoreMesh(core_axis_name="c", subcore_axis_name="s",
                               num_cores=NC, num_subcores=NS)

_sc_params = pltpu.CompilerParams(use_tc_tiling_on_sc=False)

_f32 = jnp.float32


def _lrelu(v):
    return jnp.where(v >= 0.0, v, 0.2 * v)


def _zero_stripe(zb, sp_ref, base_r):
    for k in range(ZR // ZB):
        pltpu.sync_copy(zb, sp_ref.at[pl.ds(base_r + k * ZB, ZB), :])
    rem = ZR - (ZR // ZB) * ZB
    pltpu.sync_copy(zb.at[pl.ds(0, rem)],
                    sp_ref.at[pl.ds(base_r + (ZR // ZB) * ZB, rem), :])


def _splat(v, h):
    dn = lax.GatherDimensionNumbers(offset_dims=(), collapsed_slice_dims=(0,),
                                    start_index_map=(0,))
    idx = jnp.full((16, 1), h, jnp.int32)
    return lax.gather(v, idx, dn, (1,),
                      mode=lax.GatherScatterMode.PROMISE_IN_BOUNDS)


def _make_pass1(with_cnt, lcol):
    W = 48 if with_cnt else 32
    out_type = (jax.ShapeDtypeStruct((EP, 16), _f32),
                jax.ShapeDtypeStruct((NC, NP, W), _f32))
    scratch = [pltpu.VMEM((2, 2, K), jnp.int32),
               pltpu.VMEM((2, K, 16), _f32),
               pltpu.VMEM((2, K, 16), _f32),
               pltpu.VMEM((2, K, W), _f32),
               pltpu.VMEM((ZB, W), _f32),
               pltpu.SemaphoreType.DMA,
               pltpu.SemaphoreType.DMA,
               pltpu.SemaphoreType.DMA,
               pltpu.SemaphoreType.DMA,
               pltpu.SemaphoreType.DMA,
               pltpu.SemaphoreType.DMA,
               pltpu.VMEM_SHARED((NP, W), _f32)]

    def body(src2, a16, d16p, ae, exo, acco,
             sdv2, sav, sdv, cbuf, zb, sa0, sa1, sd0, sd1, se0, se1, acc_sp):
        semA = (sa0, sa1)
        semD = (sd0, sd1)
        semE = (se0, se1)
        c = lax.axis_index("c")
        s = lax.axis_index("s")
        wid = c * NS + s

        def zb_body(i, carry):
            for k in range(W // 16):
                zb[i, pl.ds(16 * k, 16)] = jnp.zeros((16,), _f32)
            return carry
        lax.fori_loop(0, ZB, zb_body, 0)
        if with_cnt:
            def ones_body(i, carry):
                cbuf[0, i, pl.ds(32, 16)] = jnp.ones((16,), _f32)
                cbuf[1, i, pl.ds(32, 16)] = jnp.ones((16,), _f32)
                return carry
            lax.fori_loop(0, K, ones_body, 0)
        _zero_stripe(zb, acc_sp, s * ZR)
        plsc.subcore_barrier()

        def linear(j, b):
            base = (wid * CH + j) * K
            pltpu.sync_copy(src2.at[:, pl.ds(base, K)], sdv2.at[b])
            pltpu.sync_copy(ae.at[pl.ds(base, K), pl.ds(lcol, 16)],
                            cbuf.at[b, :, pl.ds(16, 16)])

        def gather_issue(b):
            pltpu.async_copy(a16.at[sdv2.at[b, 0]], sav.at[b], semA[b])
            pltpu.async_copy(d16p.at[sdv2.at[b, 1]], sdv.at[b], semD[b])

        def gather_wait(b):
            pltpu.make_async_copy(a16.at[sdv2.at[b, 0]], sav.at[b], semA[b]).wait()
            pltpu.make_async_copy(d16p.at[sdv2.at[b, 1]], sdv.at[b], semD[b]).wait()

        def compute(b):
            def ebody(e, carry2):
                v = sav[b, e] + sdv[b, e] + cbuf[b, e, pl.ds(16, 16)]
                cbuf[b, e, pl.ds(0, 16)] = jnp.exp(_lrelu(v))
                return carry2
            lax.fori_loop(0, K, ebody, 0, unroll=4)

        def ex_issue(j, b):
            base = (wid * CH + j) * K
            pltpu.async_copy(cbuf.at[b, :, pl.ds(0, 16)],
                             exo.at[pl.ds(base, K), :], semE[b])

        def ex_wait(j, b):
            base = (wid * CH + j) * K
            pltpu.make_async_copy(cbuf.at[b, :, pl.ds(0, 16)],
                                  exo.at[pl.ds(base, K), :], semE[b]).wait()

        def scatter_sync(b):
            pltpu.sync_copy(cbuf.at[b], acc_sp.at[sdv2.at[b, 1]], add=True)

        linear(0, 0)
        gather_issue(0)
        linear(1, 1)
        gather_issue(1)

        def superstep(i, carry):
            j0 = 2 * i
            j1 = j0 + 1
            gather_wait(0)

            @pl.when(i > 0)
            def _():
                ex_wait(j0 - 2, 0)
            compute(0)
            ex_issue(j0, 0)
            scatter_sync(0)

            @pl.when(i < CH // 2 - 1)
            def _():
                linear(j0 + 2, 0)
                gather_issue(0)
            gather_wait(1)

            @pl.when(i > 0)
            def _():
                ex_wait(j1 - 2, 1)
            compute(1)
            ex_issue(j1, 1)
            scatter_sync(1)

            @pl.when(i < CH // 2 - 1)
            def _():
                linear(j1 + 2, 1)
                gather_issue(1)
            return carry
        lax.fori_loop(0, CH // 2, superstep, 0)
        ex_wait(CH - 2, 0)
        ex_wait(CH - 1, 1)
        plsc.subcore_barrier()

        @pl.when(s == 0)
        def _():
            pltpu.sync_copy(acc_sp, acco.at[c])

    return pl.kernel(body, out_type=out_type, mesh=_mesh,
                     scratch_types=tuple(scratch), compiler_params=_sc_params)


_pass1_l1 = _make_pass1(True, 0)
_pass1_l2 = _make_pass1(False, 16)


def _pass2_body(src2, ex, xl, outo,
                sdv2, exv, Xv, zb, sg0, sg1, out_sp):
    semG = (sg0, sg1)
    c = lax.axis_index("c")
    s = lax.axis_index("s")
    wid = c * NS + s

    def zb_body(i, carry):
        for k in range(8):
            zb[i, pl.ds(k * 16, 16)] = jnp.zeros((16,), _f32)
        return carry
    lax.fori_loop(0, ZB, zb_body, 0)
    _zero_stripe(zb, out_sp, s * ZR)
    plsc.subcore_barrier()

    def linear(j, b):
        base = (wid * CH + j) * K
        pltpu.sync_copy(src2.at[:, pl.ds(base, K)], sdv2.at[b])
        pltpu.sync_copy(ex.at[pl.ds(base, K), :], exv.at[b])

    def gather_issue(b):
        pltpu.async_copy(xl.at[sdv2.at[b, 0]], Xv.at[b], semG[b])

    def gather_wait(b):
        pltpu.make_async_copy(xl.at[sdv2.at[b, 0]], Xv.at[b], semG[b]).wait()

    def compute(b):
        def ebody(e, carry2):
            ev = exv[b, e]
            for h in range(H):
                sl = pl.ds(h * 16, 16)
                Xv[b, e, sl] = Xv[b, e, sl] * _splat(ev, h)
            return carry2
        lax.fori_loop(0, K, ebody, 0, unroll=2)

    def scatter_sync(b):
        pltpu.sync_copy(Xv.at[b], out_sp.at[sdv2.at[b, 1]], add=True)

    linear(0, 0)
    gather_issue(0)
    linear(1, 1)
    gather_issue(1)

    def superstep(i, carry):
        j0 = 2 * i
        j1 = j0 + 1
        gather_wait(0)
        compute(0)
        scatter_sync(0)

        @pl.when(i < CH // 2 - 1)
        def _():
            linear(j0 + 2, 0)
            gather_issue(0)
        gather_wait(1)
        compute(1)
        scatter_sync(1)

        @pl.when(i < CH // 2 - 1)
        def _():
            linear(j1 + 2, 1)
            gather_issue(1)
        return carry
    lax.fori_loop(0, CH // 2, superstep, 0)
    plsc.subcore_barrier()

    @pl.when(s == 0)
    def _():
        pltpu.sync_copy(out_sp, outo.at[c])


_pass2 = pl.kernel(
    _pass2_body,
    out_type=jax.ShapeDtypeStruct((NC, NP, F), _f32),
    mesh=_mesh,
    scratch_types=(pltpu.VMEM((2, 2, K), jnp.int32),
                   pltpu.VMEM((2, K, 16), _f32),
                   pltpu.VMEM((2, K, F), _f32),
                   pltpu.VMEM((ZB, F), _f32),
                   pltpu.SemaphoreType.DMA,
                   pltpu.SemaphoreType.DMA,
                   pltpu.VMEM_SHARED((NP, F), _f32)),
    compiler_params=_sc_params)


_B = 2000
_EB = 8192


def _premix1_body(x_ref, wcat_ref, xl_ref, a_ref, d_ref):
    p = jnp.dot(x_ref[...], wcat_ref[...], preferred_element_type=_f32)
    xl_ref[...] = p[:, :F]
    a_ref[...] = p[:, F:F + 16]
    d_ref[...] = p[:, F + 16:F + 32]


def _edgepre_body(ea_ref, aecat_ref, out_ref):
    out_ref[...] = jnp.dot(ea_ref[...], aecat_ref[...],
                           preferred_element_type=_f32)


def _make_combine_body(with_cnt):
    def body(outp, acc, acc1, a16, d16, xl, rep, bvec, hraw, stats):
        step = pl.program_id(0)
        den8 = acc[0, :, 0:8] + acc[1, :, 0:8]
        es8 = acc[0, :, 16:24] + acc[1, :, 16:24]
        if with_cnt:
            cnt8 = acc[0, :, 32:40] + acc[1, :, 32:40]
        else:
            cnt8 = acc1[0, :, 32:40] + acc1[1, :, 32:40]
        al = a16[:, :8] + d16[:, :8] + es8 / jnp.maximum(cnt8, 1.0)
        exl = jnp.exp(_lrelu(al))
        dtot = den8 + exl
        o = outp[0] + outp[1]
        hr = (o + xl[...] * jnp.dot(exl, rep[...], preferred_element_type=_f32)) \
            / jnp.dot(dtot, rep[...], preferred_element_type=_f32) + bvec[...]
        hraw[...] = hr
        st = jnp.concatenate([jnp.sum(hr, axis=0, keepdims=True),
                              jnp.sum(hr * hr, axis=0, keepdims=True)], axis=0)

        @pl.when(step == 0)
        def _():
            stats[...] = st

        @pl.when(step > 0)
        def _():
            stats[...] += st
    return body


def _bn_elu(h_blk, stats):
    m = stats[0:1, :] / N
    v = stats[1:2, :] / N - m * m
    xin = (h_blk - m) / jnp.sqrt(v + 1e-5)
    return jnp.where(xin > 0, xin, jnp.exp(xin) - 1.0)


def _premix2_body(hraw, stats, wcat, xl_ref, a_ref, d_ref):
    xin = _bn_elu(hraw[...], stats[...])
    p = jnp.dot(xin, wcat[...], preferred_element_type=_f32)
    xl_ref[...] = p[:, :F]
    a_ref[...] = p[:, F:F + 16]
    d_ref[...] = p[:, F + 16:F + 32]


def _head_body(hraw, stats, wl, blv, out_ref):
    xin = _bn_elu(hraw[...], stats[...])
    out_ref[...] = jnp.dot(xin, wl[...], preferred_element_type=_f32) + blv[...]


def _node_spec(width):
    return pl.BlockSpec((_B, width), lambda i: (i, 0))


def _full_spec(shape):
    return pl.BlockSpec(shape, lambda i: tuple(0 for _ in shape))


_premix = pl.pallas_call(
    _premix1_body,
    grid=(N // _B,),
    in_specs=[_node_spec(F), _full_spec((F, F + 32))],
    out_specs=[_node_spec(F), _node_spec(16), _node_spec(16)],
    out_shape=[jax.ShapeDtypeStruct((N, F), _f32),
               jax.ShapeDtypeStruct((N, 16), _f32),
               jax.ShapeDtypeStruct((N, 16), _f32)],
)

_edgepre = pl.pallas_call(
    _edgepre_body,
    grid=(EP // _EB,),
    in_specs=[pl.BlockSpec((_EB, ED), lambda i: (i, 0)), _full_spec((ED, 32))],
    out_specs=pl.BlockSpec((_EB, 32), lambda i: (i, 0)),
    out_shape=jax.ShapeDtypeStruct((EP, 32), _f32),
)


def _make_combine(with_cnt):
    w = 48 if with_cnt else 32
    return pl.pallas_call(
        _make_combine_body(with_cnt),
        grid=(N // _B,),
        in_specs=[pl.BlockSpec((NC, _B, F), lambda i: (0, i, 0)),
                  pl.BlockSpec((NC, _B, w), lambda i: (0, i, 0)),
                  pl.BlockSpec((NC, _B, 48), lambda i: (0, i, 0)),
                  _node_spec(16), _node_spec(16), _node_spec(F),
                  _full_spec((H, F)), _full_spec((1, F))],
        out_specs=[_node_spec(F), _full_spec((2, F))],
        out_shape=[jax.ShapeDtypeStruct((N, F), _f32),
                   jax.ShapeDtypeStruct((2, F), _f32)],
    )


_combine1 = _make_combine(True)
_combine2 = _make_combine(False)

_premix_next = pl.pallas_call(
    _premix2_body,
    grid=(N // _B,),
    in_specs=[_node_spec(F), _full_spec((2, F)), _full_spec((F, F + 32))],
    out_specs=[_node_spec(F), _node_spec(16), _node_spec(16)],
    out_shape=[jax.ShapeDtypeStruct((N, F), _f32),
               jax.ShapeDtypeStruct((N, 16), _f32),
               jax.ShapeDtypeStruct((N, 16), _f32)],
)

_head = pl.pallas_call(
    _head_body,
    grid=(N // _B,),
    in_specs=[_node_spec(F), _full_spec((2, F)), _full_spec((F, OUT)),
              _full_spec((1, OUT))],
    out_specs=_node_spec(OUT),
    out_shape=jax.ShapeDtypeStruct((N, OUT), _f32),
)


def _fold(W, att):
    a = (W.reshape(W.shape[0], H, C) * att[None]).sum(-1)
    return jnp.concatenate([a, a], axis=1)


def kernel(x, edge_index, edge_attr, W1, as1, ad1, We1, ae1, b1,
           W2, as2, ad2, We2, ae2, b2, Wl, bl):
    src = edge_index[0]
    dst = edge_index[1]
    pad = EP - E
    src_p = jnp.concatenate([src, jnp.arange(pad, dtype=jnp.int32) % N])
    dst_p = jnp.concatenate([dst,
                             N + (jnp.arange(pad, dtype=jnp.int32) % (NP - N))])
    ei_p = jnp.stack([src_p, dst_p])
    ea_p = jnp.concatenate([edge_attr, jnp.zeros((pad, ED), _f32)], axis=0)

    aecat = jnp.concatenate([_fold(We1, ae1), _fold(We2, ae2)], axis=1)
    wcat1 = jnp.concatenate([W1, _fold(W1, as1), _fold(W1, ad1)], axis=1)
    wcat2 = jnp.concatenate([W2, _fold(W2, as2), _fold(W2, ad2)], axis=1)
    rep = jnp.zeros((H, F), _f32)
    rep = rep.at[jnp.repeat(jnp.arange(H), C), jnp.arange(F)].set(1.0)

    AE = _edgepre(ea_p, aecat)
    xl1, a1, d1 = _premix(x, wcat1)
    d1p = jnp.concatenate([d1, jnp.zeros((NP - N, 16), _f32)], axis=0)

    ex1, acc1 = _pass1_l1(ei_p, a1, d1p, AE)
    out1 = _pass2(ei_p, ex1, xl1)
    h1, st1 = _combine1(out1, acc1, acc1, a1, d1, xl1, rep, b1.reshape(1, F))

    xl2, a2, d2 = _premix_next(h1, st1, wcat2)
    d2p = jnp.concatenate([d2, jnp.zeros((NP - N, 16), _f32)], axis=0)

    ex2, acc2 = _pass1_l2(ei_p, a2, d2p, AE)
    out2 = _pass2(ei_p, ex2, xl2)
    h2, st2 = _combine2(out2, acc2, acc1, a2, d2, xl2, rep, b2.reshape(1, F))

    return _head(h2, st2, Wl, bl.reshape(1, OUT))


# speedup vs baseline: 1.6866x; 1.2427x over previous
import jax
import jax.numpy as jnp
from jax import lax
from jax.experimental import pallas as pl
from jax.experimental.pallas import tpu as pltpu
from jax.experimental.pallas import tpu_sc as plsc

N = 10000
E = 320000
H = 8
C = 16
F = 128
ED = 16
OUT = 16

NC = 2
NS = 16
NW = NC * NS
K = 128
EP = 327680
EW = EP // NW
CH = EW // K
NP = N + 16
ZR = NP // NS
ZB = 64

_mesh = plsc.VectorSubcoreMesh(core_axis_name="c", subcore_axis_name="s",
                               num_cores=NC, num_subcores=NS)

_sc_params = pltpu.CompilerParams(use_tc_tiling_on_sc=False)

_f32 = jnp.float32


def _lrelu(v):
    return jnp.where(v >= 0.0, v, 0.2 * v)


def _zero_stripe(zb, sp_ref, base_r):
    for k in range(ZR // ZB):
        pltpu.sync_copy(zb, sp_ref.at[pl.ds(base_r + k * ZB, ZB), :])
    rem = ZR - (ZR // ZB) * ZB
    pltpu.sync_copy(zb.at[pl.ds(0, rem)],
                    sp_ref.at[pl.ds(base_r + (ZR // ZB) * ZB, rem), :])


def _splat(v, h):
    dn = lax.GatherDimensionNumbers(offset_dims=(), collapsed_slice_dims=(0,),
                                    start_index_map=(0,))
    idx = jnp.full((16, 1), h, jnp.int32)
    return lax.gather(v, idx, dn, (1,),
                      mode=lax.GatherScatterMode.PROMISE_IN_BOUNDS)


def _make_pass1(with_cnt, lcol):
    W = 48 if with_cnt else 32
    out_type = (jax.ShapeDtypeStruct((EP, 16), _f32),
                jax.ShapeDtypeStruct((NC, NP, W), _f32))
    scratch = [pltpu.VMEM((2, CH, K), jnp.int32),
               pltpu.VMEM((2, K, 16), _f32),
               pltpu.VMEM((2, K, 16), _f32),
               pltpu.VMEM((2, K, W), _f32),
               pltpu.VMEM((ZB, W), _f32),
               pltpu.SemaphoreType.DMA,
               pltpu.SemaphoreType.DMA,
               pltpu.SemaphoreType.DMA,
               pltpu.SemaphoreType.DMA,
               pltpu.SemaphoreType.DMA,
               pltpu.SemaphoreType.DMA,
               pltpu.SemaphoreType.DMA,
               pltpu.SemaphoreType.DMA,
               pltpu.VMEM_SHARED((NP, W), _f32)]

    def body(ei3, a16, d16p, ae, exo, acco,
             idxv, sav, sdv, cbuf, zb, sa0, sa1, sd0, sd1, se0, se1,
             sb0, sb1, acc_sp):
        semA = (sa0, sa1)
        semD = (sd0, sd1)
        semE = (se0, se1)
        semB = (sb0, sb1)
        c = lax.axis_index("c")
        s = lax.axis_index("s")
        wid = c * NS + s

        def zb_body(i, carry):
            for k in range(W // 16):
                zb[i, pl.ds(16 * k, 16)] = jnp.zeros((16,), _f32)
            return carry
        lax.fori_loop(0, ZB, zb_body, 0)
        if with_cnt:
            def ones_body(i, carry):
                cbuf[0, i, pl.ds(32, 16)] = jnp.ones((16,), _f32)
                cbuf[1, i, pl.ds(32, 16)] = jnp.ones((16,), _f32)
                return carry
            lax.fori_loop(0, K, ones_body, 0)
        pltpu.sync_copy(ei3.at[:, wid], idxv)
        _zero_stripe(zb, acc_sp, s * ZR)
        plsc.subcore_barrier()

        def ae_issue(j, b):
            base = (wid * CH + j) * K
            pltpu.async_copy(ae.at[pl.ds(base, K), pl.ds(lcol, 16)],
                             cbuf.at[b, :, pl.ds(16, 16)], semB[b])

        def ae_wait(j, b):
            base = (wid * CH + j) * K
            pltpu.make_async_copy(ae.at[pl.ds(base, K), pl.ds(lcol, 16)],
                                  cbuf.at[b, :, pl.ds(16, 16)], semB[b]).wait()

        def gather_issue(j, b):
            pltpu.async_copy(a16.at[idxv.at[0, j]], sav.at[b], semA[b])
            pltpu.async_copy(d16p.at[idxv.at[1, j]], sdv.at[b], semD[b])

        def gather_wait(j, b):
            pltpu.make_async_copy(a16.at[idxv.at[0, j]], sav.at[b], semA[b]).wait()
            pltpu.make_async_copy(d16p.at[idxv.at[1, j]], sdv.at[b], semD[b]).wait()

        def compute(b):
            def ebody(e, carry2):
                v = sav[b, e] + sdv[b, e] + cbuf[b, e, pl.ds(16, 16)]
                cbuf[b, e, pl.ds(0, 16)] = jnp.exp(_lrelu(v))
                return carry2
            lax.fori_loop(0, K, ebody, 0, unroll=4)

        def ex_issue(j, b):
            base = (wid * CH + j) * K
            pltpu.async_copy(cbuf.at[b, :, pl.ds(0, 16)],
                             exo.at[pl.ds(base, K), :], semE[b])

        def ex_wait(j, b):
            base = (wid * CH + j) * K
            pltpu.make_async_copy(cbuf.at[b, :, pl.ds(0, 16)],
                                  exo.at[pl.ds(base, K), :], semE[b]).wait()

        def scatter_sync(j, b):
            pltpu.sync_copy(cbuf.at[b], acc_sp.at[idxv.at[1, j]], add=True)

        ae_issue(0, 0)
        gather_issue(0, 0)
        ae_issue(1, 1)
        gather_issue(1, 1)

        def superstep(i, carry):
            for b in range(2):
                j = 2 * i + b
                gather_wait(j, b)

                @pl.when(i > 0)
                def _():
                    ex_wait(j - 2, b)
                ae_wait(j, b)
                compute(b)
                ex_issue(j, b)
                scatter_sync(j, b)

                @pl.when(i < CH // 2 - 1)
                def _():
                    ae_issue(j + 2, b)
                    gather_issue(j + 2, b)
            return carry
        lax.fori_loop(0, CH // 2, superstep, 0)
        ex_wait(CH - 2, 0)
        ex_wait(CH - 1, 1)
        plsc.subcore_barrier()

        @pl.when(s == 0)
        def _():
            pltpu.sync_copy(acc_sp, acco.at[c])

    return pl.kernel(body, out_type=out_type, mesh=_mesh,
                     scratch_types=tuple(scratch), compiler_params=_sc_params)


_pass1_l1 = _make_pass1(True, 0)
_pass1_l2 = _make_pass1(False, 16)


def _pass2_body(ei3, ex, xl, outo,
                idx3, exv, Xv, zb, si0, si1, si2, sx0, sx1, sx2, sg0, sg1,
                out_sp):
    semI = (si0, si1, si2)
    semX = (sx0, sx1, sx2)
    semG = (sg0, sg1)
    c = lax.axis_index("c")
    s = lax.axis_index("s")
    wid = c * NS + s

    def zb_body(i, carry):
        for k in range(8):
            zb[i, pl.ds(k * 16, 16)] = jnp.zeros((16,), _f32)
        return carry
    lax.fori_loop(0, ZB, zb_body, 0)
    _zero_stripe(zb, out_sp, s * ZR)
    plsc.subcore_barrier()

    def idx_issue(j, t):
        pltpu.async_copy(ei3.at[:, wid, j], idx3.at[t], semI[t])

    def idx_wait(j, t):
        pltpu.make_async_copy(ei3.at[:, wid, j], idx3.at[t], semI[t]).wait()

    def ex_issue(j, t):
        base = (wid * CH + j) * K
        pltpu.async_copy(ex.at[pl.ds(base, K), :], exv.at[t], semX[t])

    def ex_wait(j, t):
        base = (wid * CH + j) * K
        pltpu.make_async_copy(ex.at[pl.ds(base, K), :], exv.at[t], semX[t]).wait()

    def gather_issue(t, b):
        pltpu.async_copy(xl.at[idx3.at[t, 0]], Xv.at[b], semG[b])

    def gather_wait(t, b):
        pltpu.make_async_copy(xl.at[idx3.at[t, 0]], Xv.at[b], semG[b]).wait()

    def compute(b, t):
        def ebody(e, carry2):
            ev = exv[t, e]
            for h in range(H):
                sl = pl.ds(h * 16, 16)
                Xv[b, e, sl] = Xv[b, e, sl] * _splat(ev, h)
            return carry2
        lax.fori_loop(0, K, ebody, 0, unroll=2)

    def scatter_sync(b, t):
        pltpu.sync_copy(Xv.at[b], out_sp.at[idx3.at[t, 1]], add=True)

    for t in range(3):
        idx_issue(t, t)
        ex_issue(t, t)
    idx_wait(0, 0)
    gather_issue(0, 0)
    idx_wait(1, 1)
    gather_issue(1, 1)

    def superstep(i, carry):
        for r in range(6):
            j = 6 * i + r
            b = r % 2
            t = r % 3
            gather_wait(t, b)
            ex_wait(j, t)
            compute(b, t)
            scatter_sync(b, t)

            @pl.when(j + 3 < CH)
            def _():
                idx_issue(j + 3, t)
                ex_issue(j + 3, t)

            @pl.when(j + 2 < CH)
            def _():
                idx_wait(j + 2, (r + 2) % 3)
                gather_issue((r + 2) % 3, b)
        return carry
    lax.fori_loop(0, CH // 6, superstep, 0)
    for j, b in ((CH - 2, 0), (CH - 1, 1)):
        t = b
        gather_wait(t, b)
        ex_wait(j, t)
        compute(b, t)
        scatter_sync(b, t)
    plsc.subcore_barrier()

    @pl.when(s == 0)
    def _():
        pltpu.sync_copy(out_sp, outo.at[c])


_pass2 = pl.kernel(
    _pass2_body,
    out_type=jax.ShapeDtypeStruct((NC, NP, F), _f32),
    mesh=_mesh,
    scratch_types=(pltpu.VMEM((3, 2, K), jnp.int32),
                   pltpu.VMEM((3, K, 16), _f32),
                   pltpu.VMEM((2, K, F), _f32),
                   pltpu.VMEM((ZB, F), _f32),
                   pltpu.SemaphoreType.DMA,
                   pltpu.SemaphoreType.DMA,
                   pltpu.SemaphoreType.DMA,
                   pltpu.SemaphoreType.DMA,
                   pltpu.SemaphoreType.DMA,
                   pltpu.SemaphoreType.DMA,
                   pltpu.SemaphoreType.DMA,
                   pltpu.SemaphoreType.DMA,
                   pltpu.VMEM_SHARED((NP, F), _f32)),
    compiler_params=_sc_params)


_B = 2000
_EB = 8192


def _premix1_body(x_ref, wcat_ref, xl_ref, a_ref, d_ref):
    p = jnp.dot(x_ref[...], wcat_ref[...], preferred_element_type=_f32)
    xl_ref[...] = p[:, :F]
    a_ref[...] = p[:, F:F + 16]
    d_ref[...] = p[:, F + 16:F + 32]


def _edgepre_body(ea_ref, aecat_ref, out_ref):
    out_ref[...] = jnp.dot(ea_ref[...], aecat_ref[...],
                           preferred_element_type=_f32)


def _make_combine_body(with_cnt):
    def body(outp, acc, acc1, a16, d16, xl, rep, bvec, hraw, stats):
        step = pl.program_id(0)
        den8 = acc[0, :, 0:8] + acc[1, :, 0:8]
        es8 = acc[0, :, 16:24] + acc[1, :, 16:24]
        if with_cnt:
            cnt8 = acc[0, :, 32:40] + acc[1, :, 32:40]
        else:
            cnt8 = acc1[0, :, 32:40] + acc1[1, :, 32:40]
        al = a16[:, :8] + d16[:, :8] + es8 / jnp.maximum(cnt8, 1.0)
        exl = jnp.exp(_lrelu(al))
        dtot = den8 + exl
        o = outp[0] + outp[1]
        hr = (o + xl[...] * jnp.dot(exl, rep[...], preferred_element_type=_f32)) \
            / jnp.dot(dtot, rep[...], preferred_element_type=_f32) + bvec[...]
        hraw[...] = hr
        st = jnp.concatenate([jnp.sum(hr, axis=0, keepdims=True),
                              jnp.sum(hr * hr, axis=0, keepdims=True)], axis=0)

        @pl.when(step == 0)
        def _():
            stats[...] = st

        @pl.when(step > 0)
        def _():
            stats[...] += st
    return body


def _bn_elu(h_blk, stats):
    m = stats[0:1, :] / N
    v = stats[1:2, :] / N - m * m
    xin = (h_blk - m) / jnp.sqrt(v + 1e-5)
    return jnp.where(xin > 0, xin, jnp.exp(xin) - 1.0)


def _premix2_body(hraw, stats, wcat, xl_ref, a_ref, d_ref):
    xin = _bn_elu(hraw[...], stats[...])
    p = jnp.dot(xin, wcat[...], preferred_element_type=_f32)
    xl_ref[...] = p[:, :F]
    a_ref[...] = p[:, F:F + 16]
    d_ref[...] = p[:, F + 16:F + 32]


def _head_body(hraw, stats, wl, blv, out_ref):
    xin = _bn_elu(hraw[...], stats[...])
    out_ref[...] = jnp.dot(xin, wl[...], preferred_element_type=_f32) + blv[...]


def _node_spec(width):
    return pl.BlockSpec((_B, width), lambda i: (i, 0))


def _full_spec(shape):
    return pl.BlockSpec(shape, lambda i: tuple(0 for _ in shape))


_premix = pl.pallas_call(
    _premix1_body,
    grid=(N // _B,),
    in_specs=[_node_spec(F), _full_spec((F, F + 32))],
    out_specs=[_node_spec(F), _node_spec(16), _node_spec(16)],
    out_shape=[jax.ShapeDtypeStruct((N, F), _f32),
               jax.ShapeDtypeStruct((N, 16), _f32),
               jax.ShapeDtypeStruct((N, 16), _f32)],
)

_edgepre = pl.pallas_call(
    _edgepre_body,
    grid=(EP // _EB,),
    in_specs=[pl.BlockSpec((_EB, ED), lambda i: (i, 0)), _full_spec((ED, 32))],
    out_specs=pl.BlockSpec((_EB, 32), lambda i: (i, 0)),
    out_shape=jax.ShapeDtypeStruct((EP, 32), _f32),
)


def _make_combine(with_cnt):
    w = 48 if with_cnt else 32
    return pl.pallas_call(
        _make_combine_body(with_cnt),
        grid=(N // _B,),
        in_specs=[pl.BlockSpec((NC, _B, F), lambda i: (0, i, 0)),
                  pl.BlockSpec((NC, _B, w), lambda i: (0, i, 0)),
                  pl.BlockSpec((NC, _B, 48), lambda i: (0, i, 0)),
                  _node_spec(16), _node_spec(16), _node_spec(F),
                  _full_spec((H, F)), _full_spec((1, F))],
        out_specs=[_node_spec(F), _full_spec((2, F))],
        out_shape=[jax.ShapeDtypeStruct((N, F), _f32),
                   jax.ShapeDtypeStruct((2, F), _f32)],
    )


_combine1 = _make_combine(True)
_combine2 = _make_combine(False)

_premix_next = pl.pallas_call(
    _premix2_body,
    grid=(N // _B,),
    in_specs=[_node_spec(F), _full_spec((2, F)), _full_spec((F, F + 32))],
    out_specs=[_node_spec(F), _node_spec(16), _node_spec(16)],
    out_shape=[jax.ShapeDtypeStruct((N, F), _f32),
               jax.ShapeDtypeStruct((N, 16), _f32),
               jax.ShapeDtypeStruct((N, 16), _f32)],
)

_head = pl.pallas_call(
    _head_body,
    grid=(N // _B,),
    in_specs=[_node_spec(F), _full_spec((2, F)), _full_spec((F, OUT)),
              _full_spec((1, OUT))],
    out_specs=_node_spec(OUT),
    out_shape=jax.ShapeDtypeStruct((N, OUT), _f32),
)


def _fold(W, att):
    a = (W.reshape(W.shape[0], H, C) * att[None]).sum(-1)
    return jnp.concatenate([a, a], axis=1)


def kernel(x, edge_index, edge_attr, W1, as1, ad1, We1, ae1, b1,
           W2, as2, ad2, We2, ae2, b2, Wl, bl):
    src = edge_index[0]
    dst = edge_index[1]
    pad = EP - E
    src_p = jnp.concatenate([src, jnp.arange(pad, dtype=jnp.int32) % N])
    dst_p = jnp.concatenate([dst,
                             N + (jnp.arange(pad, dtype=jnp.int32) % (NP - N))])
    ei3 = jnp.stack([src_p, dst_p]).reshape(2, NW, CH, K)
    ea_p = jnp.concatenate([edge_attr, jnp.zeros((pad, ED), _f32)], axis=0)

    aecat = jnp.concatenate([_fold(We1, ae1), _fold(We2, ae2)], axis=1)
    wcat1 = jnp.concatenate([W1, _fold(W1, as1), _fold(W1, ad1)], axis=1)
    wcat2 = jnp.concatenate([W2, _fold(W2, as2), _fold(W2, ad2)], axis=1)
    rep = jnp.zeros((H, F), _f32)
    rep = rep.at[jnp.repeat(jnp.arange(H), C), jnp.arange(F)].set(1.0)

    AE = _edgepre(ea_p, aecat)
    xl1, a1, d1 = _premix(x, wcat1)
    d1p = jnp.concatenate([d1, jnp.zeros((NP - N, 16), _f32)], axis=0)

    ex1, acc1 = _pass1_l1(ei3, a1, d1p, AE)
    out1 = _pass2(ei3, ex1, xl1)
    h1, st1 = _combine1(out1, acc1, acc1, a1, d1, xl1, rep, b1.reshape(1, F))

    xl2, a2, d2 = _premix_next(h1, st1, wcat2)
    d2p = jnp.concatenate([d2, jnp.zeros((NP - N, 16), _f32)], axis=0)

    ex2, acc2 = _pass1_l2(ei3, a2, d2p, AE)
    out2 = _pass2(ei3, ex2, xl2)
    h2, st2 = _combine2(out2, acc2, acc1, a2, d2, xl2, rep, b2.reshape(1, F))

    return _head(h2, st2, Wl, bl.reshape(1, OUT))


# speedup vs baseline: 1.7342x; 1.0282x over previous
import jax
import jax.numpy as jnp
from jax import lax
from jax.experimental import pallas as pl
from jax.experimental.pallas import tpu as pltpu
from jax.experimental.pallas import tpu_sc as plsc

N = 10000
E = 320000
H = 8
C = 16
F = 128
ED = 16
OUT = 16

NC = 2
NS = 16
NW = NC * NS
K = 128
EP = 327680
EW = EP // NW
CH = EW // K
NP = N + 16
ZR = NP // NS
ZB = 64

_mesh = plsc.VectorSubcoreMesh(core_axis_name="c", subcore_axis_name="s",
                               num_cores=NC, num_subcores=NS)

_sc_params = pltpu.CompilerParams(use_tc_tiling_on_sc=False)

_f32 = jnp.float32


def _lrelu(v):
    return jnp.where(v >= 0.0, v, 0.2 * v)


def _zero_stripe(zb, sp_ref, base_r):
    for k in range(ZR // ZB):
        pltpu.sync_copy(zb, sp_ref.at[pl.ds(base_r + k * ZB, ZB), :])
    rem = ZR - (ZR // ZB) * ZB
    pltpu.sync_copy(zb.at[pl.ds(0, rem)],
                    sp_ref.at[pl.ds(base_r + (ZR // ZB) * ZB, rem), :])


def _splat(v, h):
    dn = lax.GatherDimensionNumbers(offset_dims=(), collapsed_slice_dims=(0,),
                                    start_index_map=(0,))
    idx = jnp.full((16, 1), h, jnp.int32)
    return lax.gather(v, idx, dn, (1,),
                      mode=lax.GatherScatterMode.PROMISE_IN_BOUNDS)


def _make_pass1(with_cnt, lcol):
    W = 48 if with_cnt else 32
    out_type = (jax.ShapeDtypeStruct((EP, 16), _f32),
                jax.ShapeDtypeStruct((NC, NP, W), _f32))
    scratch = [pltpu.VMEM((2, CH, K), jnp.int32),
               pltpu.VMEM((2, K, 16), _f32),
               pltpu.VMEM((2, K, 16), _f32),
               pltpu.VMEM((3, K, W), _f32),
               pltpu.VMEM((ZB, W), _f32),
               pltpu.SemaphoreType.DMA,
               pltpu.SemaphoreType.DMA,
               pltpu.SemaphoreType.DMA,
               pltpu.SemaphoreType.DMA,
               pltpu.SemaphoreType.DMA,
               pltpu.SemaphoreType.DMA,
               pltpu.SemaphoreType.DMA,
               pltpu.SemaphoreType.DMA,
               pltpu.SemaphoreType.DMA,
               pltpu.SemaphoreType.DMA,
               pltpu.SemaphoreType.DMA,
               pltpu.SemaphoreType.DMA,
               pltpu.SemaphoreType.DMA,
               pltpu.VMEM_SHARED((NP, W), _f32)]

    def body(ei3, a16, d16p, ae, exo, acco,
             idxv, sav, sdv, cbuf, zb, sa0, sa1, sd0, sd1, se0, se1, se2,
             sb0, sb1, sb2, ss0, ss1, ss2, acc_sp):
        semA = (sa0, sa1)
        semD = (sd0, sd1)
        semE = (se0, se1, se2)
        semB = (sb0, sb1, sb2)
        semS = (ss0, ss1, ss2)
        c = lax.axis_index("c")
        s = lax.axis_index("s")
        wid = c * NS + s

        def zb_body(i, carry):
            for k in range(W // 16):
                zb[i, pl.ds(16 * k, 16)] = jnp.zeros((16,), _f32)
            return carry
        lax.fori_loop(0, ZB, zb_body, 0)
        if with_cnt:
            def ones_body(i, carry):
                cbuf[0, i, pl.ds(32, 16)] = jnp.ones((16,), _f32)
                cbuf[1, i, pl.ds(32, 16)] = jnp.ones((16,), _f32)
                cbuf[2, i, pl.ds(32, 16)] = jnp.ones((16,), _f32)
                return carry
            lax.fori_loop(0, K, ones_body, 0)
        pltpu.sync_copy(ei3.at[:, wid], idxv)
        _zero_stripe(zb, acc_sp, s * ZR)
        plsc.subcore_barrier()

        def ae_issue(j, u):
            base = (wid * CH + j) * K
            pltpu.async_copy(ae.at[pl.ds(base, K), pl.ds(lcol, 16)],
                             cbuf.at[u, :, pl.ds(16, 16)], semB[u])

        def ae_wait(j, u):
            base = (wid * CH + j) * K
            pltpu.make_async_copy(ae.at[pl.ds(base, K), pl.ds(lcol, 16)],
                                  cbuf.at[u, :, pl.ds(16, 16)], semB[u]).wait()

        def gather_issue(j, b):
            pltpu.async_copy(a16.at[idxv.at[0, j]], sav.at[b], semA[b])
            pltpu.async_copy(d16p.at[idxv.at[1, j]], sdv.at[b], semD[b])

        def gather_wait(j, b):
            pltpu.make_async_copy(a16.at[idxv.at[0, j]], sav.at[b], semA[b]).wait()
            pltpu.make_async_copy(d16p.at[idxv.at[1, j]], sdv.at[b], semD[b]).wait()

        def compute(b, u):
            def ebody(e, carry2):
                v = sav[b, e] + sdv[b, e] + cbuf[u, e, pl.ds(16, 16)]
                cbuf[u, e, pl.ds(0, 16)] = jnp.exp(_lrelu(v))
                return carry2
            lax.fori_loop(0, K, ebody, 0, unroll=4)

        def ex_issue(j, u):
            base = (wid * CH + j) * K
            pltpu.async_copy(cbuf.at[u, :, pl.ds(0, 16)],
                             exo.at[pl.ds(base, K), :], semE[u])

        def ex_drain(u):
            pltpu.make_async_copy(exo.at[pl.ds(0, K), :],
                                  cbuf.at[u, :, pl.ds(0, 16)], semE[u]).wait()

        def scatter_issue(j, u):
            pltpu.async_copy(cbuf.at[u], acc_sp.at[idxv.at[1, j]], semS[u],
                             add=True)

        def scatter_drain(u):
            pltpu.make_async_copy(acco.at[0, pl.ds(0, K)], cbuf.at[u],
                                  semS[u]).wait()

        ae_issue(0, 0)
        gather_issue(0, 0)
        ae_issue(1, 1)
        gather_issue(1, 1)

        def superstep(i, carry):
            for r in range(6):
                j = 6 * i + r
                b = r % 2
                u = r % 3
                u2 = (r + 2) % 3
                gather_wait(j, b)
                ae_wait(j, u)
                compute(b, u)
                ex_issue(j, u)
                scatter_issue(j, u)

                @pl.when(j + 2 < CH)
                def _():
                    @pl.when(j >= 1)
                    def _():
                        ex_drain(u2)
                        scatter_drain(u2)
                    ae_issue(j + 2, u2)
                    gather_issue(j + 2, b)
            return carry
        lax.fori_loop(0, CH // 6, superstep, 0)
        for j, b in ((CH - 2, 0), (CH - 1, 1)):
            u = b
            gather_wait(j, b)
            ae_wait(j, u)
            compute(b, u)
            ex_issue(j, u)
            scatter_issue(j, u)
        for u in (2, 0, 1):
            ex_drain(u)
            scatter_drain(u)
        plsc.subcore_barrier()

        @pl.when(s == 0)
        def _():
            pltpu.sync_copy(acc_sp, acco.at[c])

    return pl.kernel(body, out_type=out_type, mesh=_mesh,
                     scratch_types=tuple(scratch), compiler_params=_sc_params)


_pass1_l1 = _make_pass1(True, 0)
_pass1_l2 = _make_pass1(False, 16)


def _pass2_body(ei3, ex, xl, outo,
                idx3, exv, Xv, zb, si0, si1, si2, sx0, sx1, sx2, sg0, sg1,
                out_sp):
    semI = (si0, si1, si2)
    semX = (sx0, sx1, sx2)
    semG = (sg0, sg1)
    c = lax.axis_index("c")
    s = lax.axis_index("s")
    wid = c * NS + s

    def zb_body(i, carry):
        for k in range(8):
            zb[i, pl.ds(k * 16, 16)] = jnp.zeros((16,), _f32)
        return carry
    lax.fori_loop(0, ZB, zb_body, 0)
    _zero_stripe(zb, out_sp, s * ZR)
    plsc.subcore_barrier()

    def idx_issue(j, t):
        pltpu.async_copy(ei3.at[:, wid, j], idx3.at[t], semI[t])

    def idx_wait(j, t):
        pltpu.make_async_copy(ei3.at[:, wid, j], idx3.at[t], semI[t]).wait()

    def ex_issue(j, t):
        base = (wid * CH + j) * K
        pltpu.async_copy(ex.at[pl.ds(base, K), :], exv.at[t], semX[t])

    def ex_wait(j, t):
        base = (wid * CH + j) * K
        pltpu.make_async_copy(ex.at[pl.ds(base, K), :], exv.at[t], semX[t]).wait()

    def gather_issue(t, b):
        pltpu.async_copy(xl.at[idx3.at[t, 0]], Xv.at[b], semG[b])

    def gather_wait(t, b):
        pltpu.make_async_copy(xl.at[idx3.at[t, 0]], Xv.at[b], semG[b]).wait()

    def compute(b, t):
        def ebody(e, carry2):
            ev = exv[t, e]
            for h in range(H):
                sl = pl.ds(h * 16, 16)
                Xv[b, e, sl] = Xv[b, e, sl] * _splat(ev, h)
            return carry2
        lax.fori_loop(0, K, ebody, 0, unroll=2)

    def scatter_sync(b, t):
        pltpu.sync_copy(Xv.at[b], out_sp.at[idx3.at[t, 1]], add=True)

    for t in range(3):
        idx_issue(t, t)
        ex_issue(t, t)
    idx_wait(0, 0)
    gather_issue(0, 0)
    idx_wait(1, 1)
    gather_issue(1, 1)

    def superstep(i, carry):
        for r in range(6):
            j = 6 * i + r
            b = r % 2
            t = r % 3
            gather_wait(t, b)
            ex_wait(j, t)
            compute(b, t)
            scatter_sync(b, t)

            @pl.when(j + 3 < CH)
            def _():
                idx_issue(j + 3, t)
                ex_issue(j + 3, t)

            @pl.when(j + 2 < CH)
            def _():
                idx_wait(j + 2, (r + 2) % 3)
                gather_issue((r + 2) % 3, b)
        return carry
    lax.fori_loop(0, CH // 6, superstep, 0)
    for j, b in ((CH - 2, 0), (CH - 1, 1)):
        t = b
        gather_wait(t, b)
        ex_wait(j, t)
        compute(b, t)
        scatter_sync(b, t)
    plsc.subcore_barrier()

    @pl.when(s == 0)
    def _():
        pltpu.sync_copy(out_sp, outo.at[c])


_pass2 = pl.kernel(
    _pass2_body,
    out_type=jax.ShapeDtypeStruct((NC, NP, F), _f32),
    mesh=_mesh,
    scratch_types=(pltpu.VMEM((3, 2, K), jnp.int32),
                   pltpu.VMEM((3, K, 16), _f32),
                   pltpu.VMEM((2, K, F), _f32),
                   pltpu.VMEM((ZB, F), _f32),
                   pltpu.SemaphoreType.DMA,
                   pltpu.SemaphoreType.DMA,
                   pltpu.SemaphoreType.DMA,
                   pltpu.SemaphoreType.DMA,
                   pltpu.SemaphoreType.DMA,
                   pltpu.SemaphoreType.DMA,
                   pltpu.SemaphoreType.DMA,
                   pltpu.SemaphoreType.DMA,
                   pltpu.VMEM_SHARED((NP, F), _f32)),
    compiler_params=_sc_params)


_B = 2000
_EB = 8192


def _premix1_body(x_ref, wcat_ref, xl_ref, a_ref, d_ref):
    p = jnp.dot(x_ref[...], wcat_ref[...], preferred_element_type=_f32)
    xl_ref[...] = p[:, :F]
    a_ref[...] = p[:, F:F + 16]
    d_ref[...] = p[:, F + 16:F + 32]


def _edgepre_body(ea_ref, aecat_ref, out_ref):
    out_ref[...] = jnp.dot(ea_ref[...], aecat_ref[...],
                           preferred_element_type=_f32)


def _make_combine_body(with_cnt):
    def body(outp, acc, acc1, a16, d16, xl, rep, bvec, hraw, stats):
        step = pl.program_id(0)
        den8 = acc[0, :, 0:8] + acc[1, :, 0:8]
        es8 = acc[0, :, 16:24] + acc[1, :, 16:24]
        if with_cnt:
            cnt8 = acc[0, :, 32:40] + acc[1, :, 32:40]
        else:
            cnt8 = acc1[0, :, 32:40] + acc1[1, :, 32:40]
        al = a16[:, :8] + d16[:, :8] + es8 / jnp.maximum(cnt8, 1.0)
        exl = jnp.exp(_lrelu(al))
        dtot = den8 + exl
        o = outp[0] + outp[1]
        hr = (o + xl[...] * jnp.dot(exl, rep[...], preferred_element_type=_f32)) \
            / jnp.dot(dtot, rep[...], preferred_element_type=_f32) + bvec[...]
        hraw[...] = hr
        st = jnp.concatenate([jnp.sum(hr, axis=0, keepdims=True),
                              jnp.sum(hr * hr, axis=0, keepdims=True)], axis=0)

        @pl.when(step == 0)
        def _():
            stats[...] = st

        @pl.when(step > 0)
        def _():
            stats[...] += st
    return body


def _bn_elu(h_blk, stats):
    m = stats[0:1, :] / N
    v = stats[1:2, :] / N - m * m
    xin = (h_blk - m) / jnp.sqrt(v + 1e-5)
    return jnp.where(xin > 0, xin, jnp.exp(xin) - 1.0)


def _premix2_body(hraw, stats, wcat, xl_ref, a_ref, d_ref):
    xin = _bn_elu(hraw[...], stats[...])
    p = jnp.dot(xin, wcat[...], preferred_element_type=_f32)
    xl_ref[...] = p[:, :F]
    a_ref[...] = p[:, F:F + 16]
    d_ref[...] = p[:, F + 16:F + 32]


def _head_body(hraw, stats, wl, blv, out_ref):
    xin = _bn_elu(hraw[...], stats[...])
    out_ref[...] = jnp.dot(xin, wl[...], preferred_element_type=_f32) + blv[...]


def _node_spec(width):
    return pl.BlockSpec((_B, width), lambda i: (i, 0))


def _full_spec(shape):
    return pl.BlockSpec(shape, lambda i: tuple(0 for _ in shape))


_premix = pl.pallas_call(
    _premix1_body,
    grid=(N // _B,),
    in_specs=[_node_spec(F), _full_spec((F, F + 32))],
    out_specs=[_node_spec(F), _node_spec(16), _node_spec(16)],
    out_shape=[jax.ShapeDtypeStruct((N, F), _f32),
               jax.ShapeDtypeStruct((N, 16), _f32),
               jax.ShapeDtypeStruct((N, 16), _f32)],
)

_edgepre = pl.pallas_call(
    _edgepre_body,
    grid=(EP // _EB,),
    in_specs=[pl.BlockSpec((_EB, ED), lambda i: (i, 0)), _full_spec((ED, 32))],
    out_specs=pl.BlockSpec((_EB, 32), lambda i: (i, 0)),
    out_shape=jax.ShapeDtypeStruct((EP, 32), _f32),
)


def _make_combine(with_cnt):
    w = 48 if with_cnt else 32
    return pl.pallas_call(
        _make_combine_body(with_cnt),
        grid=(N // _B,),
        in_specs=[pl.BlockSpec((NC, _B, F), lambda i: (0, i, 0)),
                  pl.BlockSpec((NC, _B, w), lambda i: (0, i, 0)),
                  pl.BlockSpec((NC, _B, 48), lambda i: (0, i, 0)),
                  _node_spec(16), _node_spec(16), _node_spec(F),
                  _full_spec((H, F)), _full_spec((1, F))],
        out_specs=[_node_spec(F), _full_spec((2, F))],
        out_shape=[jax.ShapeDtypeStruct((N, F), _f32),
                   jax.ShapeDtypeStruct((2, F), _f32)],
    )


_combine1 = _make_combine(True)
_combine2 = _make_combine(False)

_premix_next = pl.pallas_call(
    _premix2_body,
    grid=(N // _B,),
    in_specs=[_node_spec(F), _full_spec((2, F)), _full_spec((F, F + 32))],
    out_specs=[_node_spec(F), _node_spec(16), _node_spec(16)],
    out_shape=[jax.ShapeDtypeStruct((N, F), _f32),
               jax.ShapeDtypeStruct((N, 16), _f32),
               jax.ShapeDtypeStruct((N, 16), _f32)],
)

_head = pl.pallas_call(
    _head_body,
    grid=(N // _B,),
    in_specs=[_node_spec(F), _full_spec((2, F)), _full_spec((F, OUT)),
              _full_spec((1, OUT))],
    out_specs=_node_spec(OUT),
    out_shape=jax.ShapeDtypeStruct((N, OUT), _f32),
)


def _fold(W, att):
    a = (W.reshape(W.shape[0], H, C) * att[None]).sum(-1)
    return jnp.concatenate([a, a], axis=1)


def kernel(x, edge_index, edge_attr, W1, as1, ad1, We1, ae1, b1,
           W2, as2, ad2, We2, ae2, b2, Wl, bl):
    src = edge_index[0]
    dst = edge_index[1]
    pad = EP - E
    src_p = jnp.concatenate([src, jnp.arange(pad, dtype=jnp.int32) % N])
    dst_p = jnp.concatenate([dst,
                             N + (jnp.arange(pad, dtype=jnp.int32) % (NP - N))])
    ei3 = jnp.stack([src_p, dst_p]).reshape(2, NW, CH, K)
    ea_p = jnp.concatenate([edge_attr, jnp.zeros((pad, ED), _f32)], axis=0)

    aecat = jnp.concatenate([_fold(We1, ae1), _fold(We2, ae2)], axis=1)
    wcat1 = jnp.concatenate([W1, _fold(W1, as1), _fold(W1, ad1)], axis=1)
    wcat2 = jnp.concatenate([W2, _fold(W2, as2), _fold(W2, ad2)], axis=1)
    rep = jnp.zeros((H, F), _f32)
    rep = rep.at[jnp.repeat(jnp.arange(H), C), jnp.arange(F)].set(1.0)

    AE = _edgepre(ea_p, aecat)
    xl1, a1, d1 = _premix(x, wcat1)
    d1p = jnp.concatenate([d1, jnp.zeros((NP - N, 16), _f32)], axis=0)

    ex1, acc1 = _pass1_l1(ei3, a1, d1p, AE)
    out1 = _pass2(ei3, ex1, xl1)
    h1, st1 = _combine1(out1, acc1, acc1, a1, d1, xl1, rep, b1.reshape(1, F))

    xl2, a2, d2 = _premix_next(h1, st1, wcat2)
    d2p = jnp.concatenate([d2, jnp.zeros((NP - N, 16), _f32)], axis=0)

    ex2, acc2 = _pass1_l2(ei3, a2, d2p, AE)
    out2 = _pass2(ei3, ex2, xl2)
    h2, st2 = _combine2(out2, acc2, acc1, a2, d2, xl2, rep, b2.reshape(1, F))

    return _head(h2, st2, Wl, bl.reshape(1, OUT))
